# baseline (speedup 1.0000x reference)
import jax
import jax.numpy as jnp
import numpy as np
from jax.experimental import pallas as pl
from jax.experimental.pallas import tpu as pltpu

N = 2048
H = 256
OUT = 512
T = 5
HEADS = 8
DH = H // HEADS
P = 16
TI = 256
NI = N // TI

_BF = jnp.bfloat16
_F32 = jnp.float32
_INV_SQRT_DH = float(1.0 / np.sqrt(DH))


def _ln(r, g, b):
    mu = jnp.mean(r, axis=1, keepdims=True)
    d = r - mu
    v = jnp.mean(d * d, axis=1, keepdims=True)
    return d * jax.lax.rsqrt(v + 1e-5) * g + b


def _dot(a, b):
    return jax.lax.dot_general(a.astype(_BF), b.astype(_BF),
                               (((1,), (0,)), ((), ())),
                               preferred_element_type=_F32)


def _qkv(h, wq_ref, bq_ref, wk_ref, bk_ref, wv_ref, bv_ref,
         q_ref, k_ref, v_ref):
    q_ref[...] = ((_dot(h, wq_ref[...]) + bq_ref[...])
                  * _INV_SQRT_DH).astype(_BF)
    k_ref[...] = (_dot(h, wk_ref[...]) + bk_ref[...]).astype(_BF)
    v_ref[...] = (_dot(h, wv_ref[...]) + bv_ref[...]).astype(_BF)


def _embed_proj_body(xr_ref, embW_ref, embb_ref, pos_ref, posW_ref, posb_ref,
                     wq_ref, bq_ref, wk_ref, bk_ref, wv_ref, bv_ref,
                     x_ref, q_ref, k_ref, v_ref):
    x = xr_ref[...] * embW_ref[...] + embb_ref[...]
    x_ref[...] = x
    h = x + _dot(pos_ref[...], posW_ref[...]) + posb_ref[...]
    _qkv(h, wq_ref, bq_ref, wk_ref, bk_ref, wv_ref, bv_ref,
         q_ref, k_ref, v_ref)


def _embed_proj(xr, embW, embb, pos, posW, posb, wq, bq, wk, bk, wv, bv):
    return pl.pallas_call(
        _embed_proj_body,
        out_shape=(
            jax.ShapeDtypeStruct((N, H), _F32),
            jax.ShapeDtypeStruct((N, H), _BF),
            jax.ShapeDtypeStruct((N, H), _BF),
            jax.ShapeDtypeStruct((N, H), _BF),
        ),
    )(xr, embW, embb, pos, posW, posb, wq, bq, wk, bk, wv, bv)


def _loc_attn_body(q_ref, k_ref, v_ref, cnt_ref, eb_ref, cidr_ref, cidc_ref,
                   o_ref):
    penalty = jnp.where(cidr_ref[0] == cidc_ref[...],
                        jnp.float32(0), jnp.float32(-1e9))
    q = q_ref[...]
    k = k_ref[...]
    v = v_ref[...]
    outs = []
    for hd in range(HEADS):
        sl = slice(hd * DH, (hd + 1) * DH)
        s = jax.lax.dot_general(q[:, sl], k[:, sl], (((1,), (1,)), ((), ())),
                                preferred_element_type=_F32)
        bias = eb_ref[hd, 0].astype(_BF) * cnt_ref[0]
        for t in range(1, T):
            bias = bias + eb_ref[hd, t].astype(_BF) * cnt_ref[t]
        s = (s + bias.astype(_F32)) + penalty
        m = jnp.max(s, axis=1, keepdims=True)
        p = jnp.exp(s - m)
        denom = jnp.sum(p, axis=1, keepdims=True)
        o = jax.lax.dot_general(p.astype(_BF), v[:, sl],
                                (((1,), (0,)), ((), ())),
                                preferred_element_type=_F32)
        outs.append((o * (1.0 / denom)).astype(_BF))
    o_ref[...] = jnp.concatenate(outs, axis=1)


def _loc_attn(q, k, v, cnt, ebT, cid_r3, cid_c):
    return pl.pallas_call(
        _loc_attn_body,
        grid=(NI,),
        in_specs=[
            pl.BlockSpec((TI, H), lambda i: (i, 0)),
            pl.BlockSpec((N, H), lambda i: (0, 0)),
            pl.BlockSpec((N, H), lambda i: (0, 0)),
            pl.BlockSpec((T, TI, N), lambda i: (0, i, 0)),
            pl.BlockSpec(memory_space=pltpu.SMEM),
            pl.BlockSpec((1, TI, 1), lambda i: (i, 0, 0)),
            pl.BlockSpec((1, N), lambda i: (0, 0)),
        ],
        out_specs=pl.BlockSpec((TI, H), lambda i: (i, 0)),
        out_shape=jax.ShapeDtypeStruct((N, H), _BF),
    )(q, k, v, cnt, ebT, cid_r3, cid_c)


def _glb_attn_body(q_ref, k_ref, v_ref, cidr_ref, cidc_ref, o_ref):
    s = jax.lax.dot_general(q_ref[...], k_ref[...], (((1,), (1,)), ((), ())),
                            preferred_element_type=_F32)
    s = jnp.where(cidr_ref[0] == cidc_ref[...], jnp.float32(-1e9), s)
    m = jnp.max(s, axis=1, keepdims=True)
    p = jnp.exp(s - m)
    denom = jnp.sum(p, axis=1, keepdims=True)
    o = jax.lax.dot_general(p.astype(_BF), v_ref[...],
                            (((1,), (0,)), ((), ())),
                            preferred_element_type=_F32)
    o_ref[...] = (o * (1.0 / denom)).astype(_BF)


def _glb_attn(q, k, v, cid_r3, cid_c):
    return pl.pallas_call(
        _glb_attn_body,
        grid=(NI,),
        in_specs=[
            pl.BlockSpec((TI, H), lambda i: (i, 0)),
            pl.BlockSpec((N, H), lambda i: (0, 0)),
            pl.BlockSpec((N, H), lambda i: (0, 0)),
            pl.BlockSpec((1, TI, 1), lambda i: (i, 0, 0)),
            pl.BlockSpec((1, N), lambda i: (0, 0)),
        ],
        out_specs=pl.BlockSpec((TI, H), lambda i: (i, 0)),
        out_shape=jax.ShapeDtypeStruct((N, H), _BF),
    )(q, k, v, cid_r3, cid_c)


def _post_proj_loc_body(o_ref, x_ref, wo_ref, bo_ref, g_ref, b_ref,
                        pos_ref, posW_ref, posb_ref,
                        wq_ref, bq_ref, wk_ref, bk_ref, wv_ref, bv_ref,
                        xn_ref, q_ref, k_ref, v_ref):
    r = x_ref[...] + _dot(o_ref[...], wo_ref[...]) + bo_ref[...]
    xn = _ln(r, g_ref[...], b_ref[...])
    xn_ref[...] = xn
    h = xn + _dot(pos_ref[...], posW_ref[...]) + posb_ref[...]
    _qkv(h, wq_ref, bq_ref, wk_ref, bk_ref, wv_ref, bv_ref,
         q_ref, k_ref, v_ref)


def _post_proj_loc(o, x, wo, bo, g, b, pos, posW, posb, wq, bq, wk, bk, wv, bv):
    return pl.pallas_call(
        _post_proj_loc_body,
        out_shape=(
            jax.ShapeDtypeStruct((N, H), _F32),
            jax.ShapeDtypeStruct((N, H), _BF),
            jax.ShapeDtypeStruct((N, H), _BF),
            jax.ShapeDtypeStruct((N, H), _BF),
        ),
    )(o, x, wo, bo, g, b, pos, posW, posb, wq, bq, wk, bk, wv, bv)


def _post_proj_glb_body(o_ref, x_ref, wo_ref, bo_ref, g_ref, b_ref,
                        wq_ref, bq_ref, wk_ref, bk_ref, wv_ref, bv_ref,
                        xn_ref, q_ref, k_ref, v_ref):
    r = x_ref[...] + _dot(o_ref[...], wo_ref[...]) + bo_ref[...]
    xn = _ln(r, g_ref[...], b_ref[...])
    xn_ref[...] = xn
    _qkv(xn, wq_ref, bq_ref, wk_ref, bk_ref, wv_ref, bv_ref,
         q_ref, k_ref, v_ref)


def _post_proj_glb(o, x, wo, bo, g, b, wq, bq, wk, bk, wv, bv):
    return pl.pallas_call(
        _post_proj_glb_body,
        out_shape=(
            jax.ShapeDtypeStruct((N, H), _F32),
            jax.ShapeDtypeStruct((N, H), _BF),
            jax.ShapeDtypeStruct((N, H), _BF),
            jax.ShapeDtypeStruct((N, H), _BF),
        ),
    )(o, x, wo, bo, g, b, wq, bq, wk, bk, wv, bv)


def _post_final_body(o_ref, x_ref, wo_ref, bo_ref, g_ref, b_ref,
                     ow_ref, ob_ref, out_ref):
    r = x_ref[...] + _dot(o_ref[...], wo_ref[...]) + bo_ref[...]
    xn = _ln(r, g_ref[...], b_ref[...])
    out_ref[...] = _dot(xn, ow_ref[...]) + ob_ref[...]


def _post_final(o, x, wo, bo, g, b, ow, ob):
    return pl.pallas_call(
        _post_final_body,
        out_shape=jax.ShapeDtypeStruct((N, OUT), _F32),
    )(o, x, wo, bo, g, b, ow, ob)


def _build_cnt(edge_index, edge_types):
    flat = (edge_types * N + edge_index[0]) * N + edge_index[1]
    cnt = jnp.zeros((T * N * N,), _BF).at[flat].add(jnp.bfloat16(1))
    return cnt.reshape(T, N, N)


def _r2(v):
    return v.reshape(1, -1)


def kernel(x, edge_index, edge_types, pos_encoding, community_ids, adj_matrix,
           emb_W, emb_b,
           loc0_posW, loc0_posb, loc0_Wq, loc0_bq, loc0_Wk, loc0_bk,
           loc0_Wv, loc0_bv, loc0_Wo, loc0_bo, loc0_ebias, loc0_lng, loc0_lnb,
           loc1_posW, loc1_posb, loc1_Wq, loc1_bq, loc1_Wk, loc1_bk,
           loc1_Wv, loc1_bv, loc1_Wo, loc1_bo, loc1_ebias, loc1_lng, loc1_lnb,
           glb0_Wq, glb0_bq, glb0_Wk, glb0_bk, glb0_Wv, glb0_bv,
           glb0_Wo, glb0_bo, glb0_lng, glb0_lnb,
           glb1_Wq, glb1_bq, glb1_Wk, glb1_bk, glb1_Wv, glb1_bv,
           glb1_Wo, glb1_bo, glb1_lng, glb1_lnb,
           out_W, out_b):
    cnt = _build_cnt(edge_index, edge_types)
    cid_r3 = community_ids.reshape(NI, TI, 1)
    cid_c = community_ids.reshape(1, N)

    x0, q, k, v = _embed_proj(x, emb_W, _r2(emb_b), pos_encoding, loc0_posW,
                              _r2(loc0_posb), loc0_Wq, _r2(loc0_bq),
                              loc0_Wk, _r2(loc0_bk), loc0_Wv, _r2(loc0_bv))
    o = _loc_attn(q, k, v, cnt, loc0_ebias.T, cid_r3, cid_c)
    x1, q, k, v = _post_proj_loc(o, x0, loc0_Wo, _r2(loc0_bo), _r2(loc0_lng),
                                 _r2(loc0_lnb), pos_encoding, loc1_posW,
                                 _r2(loc1_posb), loc1_Wq, _r2(loc1_bq),
                                 loc1_Wk, _r2(loc1_bk), loc1_Wv, _r2(loc1_bv))
    o = _loc_attn(q, k, v, cnt, loc1_ebias.T, cid_r3, cid_c)
    x2, q, k, v = _post_proj_glb(o, x1, loc1_Wo, _r2(loc1_bo), _r2(loc1_lng),
                                 _r2(loc1_lnb), glb0_Wq, _r2(glb0_bq),
                                 glb0_Wk, _r2(glb0_bk), glb0_Wv, _r2(glb0_bv))
    o = _glb_attn(q, k, v, cid_r3, cid_c)
    x3, q, k, v = _post_proj_glb(o, x2, glb0_Wo, _r2(glb0_bo), _r2(glb0_lng),
                                 _r2(glb0_lnb), glb1_Wq, _r2(glb1_bq),
                                 glb1_Wk, _r2(glb1_bk), glb1_Wv, _r2(glb1_bv))
    o = _glb_attn(q, k, v, cid_r3, cid_c)
    return _post_final(o, x3, glb1_Wo, _r2(glb1_bo), _r2(glb1_lng),
                       _r2(glb1_lnb), out_W, _r2(out_b))


# speedup vs baseline: 1.4264x; 1.4264x over previous
import jax
import jax.numpy as jnp
import numpy as np
from jax.experimental import pallas as pl
from jax.experimental.pallas import tpu as pltpu

N = 2048
H = 256
OUT = 512
T = 5
HEADS = 8
DH = H // HEADS
P = 16
TI = 256
NI = N // TI

_BF = jnp.bfloat16
_F32 = jnp.float32
_INV_SQRT_DH = float(1.0 / np.sqrt(DH))


def _ln(r, g, b):
    mu = jnp.mean(r, axis=1, keepdims=True)
    d = r - mu
    v = jnp.mean(d * d, axis=1, keepdims=True)
    return d * jax.lax.rsqrt(v + 1e-5) * g + b


def _dot(a, b):
    return jax.lax.dot_general(a.astype(_BF), b.astype(_BF),
                               (((1,), (0,)), ((), ())),
                               preferred_element_type=_F32)


def _qkv(h, wq_ref, bq_ref, wk_ref, bk_ref, wv_ref, bv_ref,
         q_ref, k_ref, v_ref):
    q_ref[...] = ((_dot(h, wq_ref[...]) + bq_ref[...])
                  * _INV_SQRT_DH).astype(_BF)
    k_ref[...] = (_dot(h, wk_ref[...]) + bk_ref[...]).astype(_BF)
    v_ref[...] = (_dot(h, wv_ref[...]) + bv_ref[...]).astype(_BF)


def _embed_proj_body(xr_ref, embW_ref, embb_ref, pos_ref, posW_ref, posb_ref,
                     wq_ref, bq_ref, wk_ref, bk_ref, wv_ref, bv_ref,
                     x_ref, q_ref, k_ref, v_ref):
    x = xr_ref[...] * embW_ref[...] + embb_ref[...]
    x_ref[...] = x
    h = x + _dot(pos_ref[...], posW_ref[...]) + posb_ref[...]
    _qkv(h, wq_ref, bq_ref, wk_ref, bk_ref, wv_ref, bv_ref,
         q_ref, k_ref, v_ref)


def _embed_proj(xr, embW, embb, pos, posW, posb, wq, bq, wk, bk, wv, bv):
    return pl.pallas_call(
        _embed_proj_body,
        out_shape=(
            jax.ShapeDtypeStruct((N, H), _F32),
            jax.ShapeDtypeStruct((N, H), _BF),
            jax.ShapeDtypeStruct((N, H), _BF),
            jax.ShapeDtypeStruct((N, H), _BF),
        ),
    )(xr, embW, embb, pos, posW, posb, wq, bq, wk, bk, wv, bv)


def _loc_attn_body(q_ref, k_ref, v_ref, cnt_ref, eb_ref, cidr_ref, cidc_ref,
                   o_ref):
    penalty = jnp.where(cidr_ref[0] == cidc_ref[...],
                        jnp.float32(0), jnp.float32(-1e9))
    cp = cnt_ref[...]
    c4 = jnp.floor(cp * (1.0 / 65536.0))
    r = cp - c4 * 65536.0
    c3 = jnp.floor(r * (1.0 / 4096.0))
    r = r - c3 * 4096.0
    c2 = jnp.floor(r * (1.0 / 256.0))
    r = r - c2 * 256.0
    c1 = jnp.floor(r * (1.0 / 16.0))
    c0 = r - c1 * 16.0
    planes = (c0, c1, c2, c3, c4)
    q = q_ref[...]
    k = k_ref[...]
    v = v_ref[...]
    outs = []
    for hd in range(HEADS):
        sl = slice(hd * DH, (hd + 1) * DH)
        s = jax.lax.dot_general(q[:, sl], k[:, sl], (((1,), (1,)), ((), ())),
                                preferred_element_type=_F32)
        bias = eb_ref[hd, 0] * planes[0]
        for t in range(1, T):
            bias = bias + eb_ref[hd, t] * planes[t]
        s = (s + bias) + penalty
        m = jnp.max(s, axis=1, keepdims=True)
        p = jnp.exp(s - m)
        denom = jnp.sum(p, axis=1, keepdims=True)
        o = jax.lax.dot_general(p.astype(_BF), v[:, sl],
                                (((1,), (0,)), ((), ())),
                                preferred_element_type=_F32)
        outs.append((o * (1.0 / denom)).astype(_BF))
    o_ref[...] = jnp.concatenate(outs, axis=1)


def _loc_attn(q, k, v, cnt, ebT, cid_r3, cid_c):
    return pl.pallas_call(
        _loc_attn_body,
        grid=(NI,),
        in_specs=[
            pl.BlockSpec((TI, H), lambda i: (i, 0)),
            pl.BlockSpec((N, H), lambda i: (0, 0)),
            pl.BlockSpec((N, H), lambda i: (0, 0)),
            pl.BlockSpec((TI, N), lambda i: (i, 0)),
            pl.BlockSpec(memory_space=pltpu.SMEM),
            pl.BlockSpec((1, TI, 1), lambda i: (i, 0, 0)),
            pl.BlockSpec((1, N), lambda i: (0, 0)),
        ],
        out_specs=pl.BlockSpec((TI, H), lambda i: (i, 0)),
        out_shape=jax.ShapeDtypeStruct((N, H), _BF),
    )(q, k, v, cnt, ebT, cid_r3, cid_c)


def _glb_attn_body(q_ref, k_ref, v_ref, cidr_ref, cidc_ref, o_ref):
    s = jax.lax.dot_general(q_ref[...], k_ref[...], (((1,), (1,)), ((), ())),
                            preferred_element_type=_F32)
    s = jnp.where(cidr_ref[0] == cidc_ref[...], jnp.float32(-1e9), s)
    m = jnp.max(s, axis=1, keepdims=True)
    p = jnp.exp(s - m)
    denom = jnp.sum(p, axis=1, keepdims=True)
    o = jax.lax.dot_general(p.astype(_BF), v_ref[...],
                            (((1,), (0,)), ((), ())),
                            preferred_element_type=_F32)
    o_ref[...] = (o * (1.0 / denom)).astype(_BF)


def _glb_attn(q, k, v, cid_r3, cid_c):
    return pl.pallas_call(
        _glb_attn_body,
        grid=(NI,),
        in_specs=[
            pl.BlockSpec((TI, H), lambda i: (i, 0)),
            pl.BlockSpec((N, H), lambda i: (0, 0)),
            pl.BlockSpec((N, H), lambda i: (0, 0)),
            pl.BlockSpec((1, TI, 1), lambda i: (i, 0, 0)),
            pl.BlockSpec((1, N), lambda i: (0, 0)),
        ],
        out_specs=pl.BlockSpec((TI, H), lambda i: (i, 0)),
        out_shape=jax.ShapeDtypeStruct((N, H), _BF),
    )(q, k, v, cid_r3, cid_c)


def _post_proj_loc_body(o_ref, x_ref, wo_ref, bo_ref, g_ref, b_ref,
                        pos_ref, posW_ref, posb_ref,
                        wq_ref, bq_ref, wk_ref, bk_ref, wv_ref, bv_ref,
                        xn_ref, q_ref, k_ref, v_ref):
    r = x_ref[...] + _dot(o_ref[...], wo_ref[...]) + bo_ref[...]
    xn = _ln(r, g_ref[...], b_ref[...])
    xn_ref[...] = xn
    h = xn + _dot(pos_ref[...], posW_ref[...]) + posb_ref[...]
    _qkv(h, wq_ref, bq_ref, wk_ref, bk_ref, wv_ref, bv_ref,
         q_ref, k_ref, v_ref)


def _post_proj_loc(o, x, wo, bo, g, b, pos, posW, posb, wq, bq, wk, bk, wv, bv):
    return pl.pallas_call(
        _post_proj_loc_body,
        out_shape=(
            jax.ShapeDtypeStruct((N, H), _F32),
            jax.ShapeDtypeStruct((N, H), _BF),
            jax.ShapeDtypeStruct((N, H), _BF),
            jax.ShapeDtypeStruct((N, H), _BF),
        ),
    )(o, x, wo, bo, g, b, pos, posW, posb, wq, bq, wk, bk, wv, bv)


def _post_proj_glb_body(o_ref, x_ref, wo_ref, bo_ref, g_ref, b_ref,
                        wq_ref, bq_ref, wk_ref, bk_ref, wv_ref, bv_ref,
                        xn_ref, q_ref, k_ref, v_ref):
    r = x_ref[...] + _dot(o_ref[...], wo_ref[...]) + bo_ref[...]
    xn = _ln(r, g_ref[...], b_ref[...])
    xn_ref[...] = xn
    _qkv(xn, wq_ref, bq_ref, wk_ref, bk_ref, wv_ref, bv_ref,
         q_ref, k_ref, v_ref)


def _post_proj_glb(o, x, wo, bo, g, b, wq, bq, wk, bk, wv, bv):
    return pl.pallas_call(
        _post_proj_glb_body,
        out_shape=(
            jax.ShapeDtypeStruct((N, H), _F32),
            jax.ShapeDtypeStruct((N, H), _BF),
            jax.ShapeDtypeStruct((N, H), _BF),
            jax.ShapeDtypeStruct((N, H), _BF),
        ),
    )(o, x, wo, bo, g, b, wq, bq, wk, bk, wv, bv)


def _post_final_body(o_ref, x_ref, wo_ref, bo_ref, g_ref, b_ref,
                     ow_ref, ob_ref, out_ref):
    r = x_ref[...] + _dot(o_ref[...], wo_ref[...]) + bo_ref[...]
    xn = _ln(r, g_ref[...], b_ref[...])
    out_ref[...] = _dot(xn, ow_ref[...]) + ob_ref[...]


def _post_final(o, x, wo, bo, g, b, ow, ob):
    return pl.pallas_call(
        _post_final_body,
        out_shape=jax.ShapeDtypeStruct((N, OUT), _F32),
    )(o, x, wo, bo, g, b, ow, ob)


def _build_cnt(edge_index, edge_types):
    flat = edge_index[0] * N + edge_index[1]
    vals = jnp.take(jnp.array([1.0, 16.0, 256.0, 4096.0, 65536.0], _F32),
                    edge_types)
    cnt = jnp.zeros((N * N,), _F32).at[flat].add(vals)
    return cnt.reshape(N, N)


def _r2(v):
    return v.reshape(1, -1)


def kernel(x, edge_index, edge_types, pos_encoding, community_ids, adj_matrix,
           emb_W, emb_b,
           loc0_posW, loc0_posb, loc0_Wq, loc0_bq, loc0_Wk, loc0_bk,
           loc0_Wv, loc0_bv, loc0_Wo, loc0_bo, loc0_ebias, loc0_lng, loc0_lnb,
           loc1_posW, loc1_posb, loc1_Wq, loc1_bq, loc1_Wk, loc1_bk,
           loc1_Wv, loc1_bv, loc1_Wo, loc1_bo, loc1_ebias, loc1_lng, loc1_lnb,
           glb0_Wq, glb0_bq, glb0_Wk, glb0_bk, glb0_Wv, glb0_bv,
           glb0_Wo, glb0_bo, glb0_lng, glb0_lnb,
           glb1_Wq, glb1_bq, glb1_Wk, glb1_bk, glb1_Wv, glb1_bv,
           glb1_Wo, glb1_bo, glb1_lng, glb1_lnb,
           out_W, out_b):
    cnt = _build_cnt(edge_index, edge_types)
    cid_r3 = community_ids.reshape(NI, TI, 1)
    cid_c = community_ids.reshape(1, N)

    x0, q, k, v = _embed_proj(x, emb_W, _r2(emb_b), pos_encoding, loc0_posW,
                              _r2(loc0_posb), loc0_Wq, _r2(loc0_bq),
                              loc0_Wk, _r2(loc0_bk), loc0_Wv, _r2(loc0_bv))
    o = _loc_attn(q, k, v, cnt, loc0_ebias.T, cid_r3, cid_c)
    x1, q, k, v = _post_proj_loc(o, x0, loc0_Wo, _r2(loc0_bo), _r2(loc0_lng),
                                 _r2(loc0_lnb), pos_encoding, loc1_posW,
                                 _r2(loc1_posb), loc1_Wq, _r2(loc1_bq),
                                 loc1_Wk, _r2(loc1_bk), loc1_Wv, _r2(loc1_bv))
    o = _loc_attn(q, k, v, cnt, loc1_ebias.T, cid_r3, cid_c)
    x2, q, k, v = _post_proj_glb(o, x1, loc1_Wo, _r2(loc1_bo), _r2(loc1_lng),
                                 _r2(loc1_lnb), glb0_Wq, _r2(glb0_bq),
                                 glb0_Wk, _r2(glb0_bk), glb0_Wv, _r2(glb0_bv))
    o = _glb_attn(q, k, v, cid_r3, cid_c)
    x3, q, k, v = _post_proj_glb(o, x2, glb0_Wo, _r2(glb0_bo), _r2(glb0_lng),
                                 _r2(glb0_lnb), glb1_Wq, _r2(glb1_bq),
                                 glb1_Wk, _r2(glb1_bk), glb1_Wv, _r2(glb1_bv))
    o = _glb_attn(q, k, v, cid_r3, cid_c)
    return _post_final(o, x3, glb1_Wo, _r2(glb1_bo), _r2(glb1_lng),
                       _r2(glb1_lnb), out_W, _r2(out_b))


# speedup vs baseline: 1.5113x; 1.0595x over previous
import dataclasses
import functools

import jax
import jax.numpy as jnp
import numpy as np
from jax import lax
from jax.experimental import pallas as pl
from jax.experimental.pallas import tpu as pltpu
from jax.experimental.pallas import tpu_sc as plsc

N = 2048
H = 256
OUT = 512
T = 5
HEADS = 8
DH = H // HEADS
P = 16
TI = 256
NI = N // TI

_BF = jnp.bfloat16
_F32 = jnp.float32
_INV_SQRT_DH = float(1.0 / np.sqrt(DH))


def _ln(r, g, b):
    mu = jnp.mean(r, axis=1, keepdims=True)
    d = r - mu
    v = jnp.mean(d * d, axis=1, keepdims=True)
    return d * jax.lax.rsqrt(v + 1e-5) * g + b


def _dot(a, b):
    return jax.lax.dot_general(a.astype(_BF), b.astype(_BF),
                               (((1,), (0,)), ((), ())),
                               preferred_element_type=_F32)


def _qkv(h, wq_ref, bq_ref, wk_ref, bk_ref, wv_ref, bv_ref,
         q_ref, k_ref, v_ref):
    q_ref[...] = ((_dot(h, wq_ref[...]) + bq_ref[...])
                  * _INV_SQRT_DH).astype(_BF)
    k_ref[...] = (_dot(h, wk_ref[...]) + bk_ref[...]).astype(_BF)
    v_ref[...] = (_dot(h, wv_ref[...]) + bv_ref[...]).astype(_BF)


def _embed_proj_body(xr_ref, embW_ref, embb_ref, pos_ref, posW_ref, posb_ref,
                     wq_ref, bq_ref, wk_ref, bk_ref, wv_ref, bv_ref,
                     x_ref, q_ref, k_ref, v_ref):
    x = xr_ref[...] * embW_ref[...] + embb_ref[...]
    x_ref[...] = x
    h = x + _dot(pos_ref[...], posW_ref[...]) + posb_ref[...]
    _qkv(h, wq_ref, bq_ref, wk_ref, bk_ref, wv_ref, bv_ref,
         q_ref, k_ref, v_ref)


def _embed_proj(xr, embW, embb, pos, posW, posb, wq, bq, wk, bk, wv, bv):
    return pl.pallas_call(
        _embed_proj_body,
        out_shape=(
            jax.ShapeDtypeStruct((N, H), _F32),
            jax.ShapeDtypeStruct((N, H), _BF),
            jax.ShapeDtypeStruct((N, H), _BF),
            jax.ShapeDtypeStruct((N, H), _BF),
        ),
    )(xr, embW, embb, pos, posW, posb, wq, bq, wk, bk, wv, bv)


def _loc_attn_body(q_ref, k_ref, v_ref, cnt_ref, eb_ref, cidr_ref, cidc_ref,
                   o_ref):
    penalty = jnp.where(cidr_ref[0] == cidc_ref[...],
                        jnp.float32(0), jnp.float32(-1e9))
    cp = cnt_ref[...]
    c4 = jnp.floor(cp * (1.0 / 65536.0))
    r = cp - c4 * 65536.0
    c3 = jnp.floor(r * (1.0 / 4096.0))
    r = r - c3 * 4096.0
    c2 = jnp.floor(r * (1.0 / 256.0))
    r = r - c2 * 256.0
    c1 = jnp.floor(r * (1.0 / 16.0))
    c0 = r - c1 * 16.0
    planes = (c0, c1, c2, c3, c4)
    q = q_ref[...]
    k = k_ref[...]
    v = v_ref[...]
    outs = []
    for hd in range(HEADS):
        sl = slice(hd * DH, (hd + 1) * DH)
        s = jax.lax.dot_general(q[:, sl], k[:, sl], (((1,), (1,)), ((), ())),
                                preferred_element_type=_F32)
        bias = eb_ref[hd, 0] * planes[0]
        for t in range(1, T):
            bias = bias + eb_ref[hd, t] * planes[t]
        s = (s + bias) + penalty
        m = jnp.max(s, axis=1, keepdims=True)
        p = jnp.exp(s - m)
        denom = jnp.sum(p, axis=1, keepdims=True)
        o = jax.lax.dot_general(p.astype(_BF), v[:, sl],
                                (((1,), (0,)), ((), ())),
                                preferred_element_type=_F32)
        outs.append((o * (1.0 / denom)).astype(_BF))
    o_ref[...] = jnp.concatenate(outs, axis=1)


def _loc_attn(q, k, v, cnt, ebT, cid_r3, cid_c):
    return pl.pallas_call(
        _loc_attn_body,
        grid=(NI,),
        in_specs=[
            pl.BlockSpec((TI, H), lambda i: (i, 0)),
            pl.BlockSpec((N, H), lambda i: (0, 0)),
            pl.BlockSpec((N, H), lambda i: (0, 0)),
            pl.BlockSpec((TI, N), lambda i: (i, 0)),
            pl.BlockSpec(memory_space=pltpu.SMEM),
            pl.BlockSpec((1, TI, 1), lambda i: (i, 0, 0)),
            pl.BlockSpec((1, N), lambda i: (0, 0)),
        ],
        out_specs=pl.BlockSpec((TI, H), lambda i: (i, 0)),
        out_shape=jax.ShapeDtypeStruct((N, H), _BF),
    )(q, k, v, cnt, ebT, cid_r3, cid_c)


def _glb_attn_body(q_ref, k_ref, v_ref, cidr_ref, cidc_ref, o_ref):
    s = jax.lax.dot_general(q_ref[...], k_ref[...], (((1,), (1,)), ((), ())),
                            preferred_element_type=_F32)
    s = jnp.where(cidr_ref[0] == cidc_ref[...], jnp.float32(-1e9), s)
    m = jnp.max(s, axis=1, keepdims=True)
    p = jnp.exp(s - m)
    denom = jnp.sum(p, axis=1, keepdims=True)
    o = jax.lax.dot_general(p.astype(_BF), v_ref[...],
                            (((1,), (0,)), ((), ())),
                            preferred_element_type=_F32)
    o_ref[...] = (o * (1.0 / denom)).astype(_BF)


def _glb_attn(q, k, v, cid_r3, cid_c):
    return pl.pallas_call(
        _glb_attn_body,
        grid=(NI,),
        in_specs=[
            pl.BlockSpec((TI, H), lambda i: (i, 0)),
            pl.BlockSpec((N, H), lambda i: (0, 0)),
            pl.BlockSpec((N, H), lambda i: (0, 0)),
            pl.BlockSpec((1, TI, 1), lambda i: (i, 0, 0)),
            pl.BlockSpec((1, N), lambda i: (0, 0)),
        ],
        out_specs=pl.BlockSpec((TI, H), lambda i: (i, 0)),
        out_shape=jax.ShapeDtypeStruct((N, H), _BF),
    )(q, k, v, cid_r3, cid_c)


def _post_proj_loc_body(o_ref, x_ref, wo_ref, bo_ref, g_ref, b_ref,
                        pos_ref, posW_ref, posb_ref,
                        wq_ref, bq_ref, wk_ref, bk_ref, wv_ref, bv_ref,
                        xn_ref, q_ref, k_ref, v_ref):
    r = x_ref[...] + _dot(o_ref[...], wo_ref[...]) + bo_ref[...]
    xn = _ln(r, g_ref[...], b_ref[...])
    xn_ref[...] = xn
    h = xn + _dot(pos_ref[...], posW_ref[...]) + posb_ref[...]
    _qkv(h, wq_ref, bq_ref, wk_ref, bk_ref, wv_ref, bv_ref,
         q_ref, k_ref, v_ref)


def _post_proj_loc(o, x, wo, bo, g, b, pos, posW, posb, wq, bq, wk, bk, wv, bv):
    return pl.pallas_call(
        _post_proj_loc_body,
        out_shape=(
            jax.ShapeDtypeStruct((N, H), _F32),
            jax.ShapeDtypeStruct((N, H), _BF),
            jax.ShapeDtypeStruct((N, H), _BF),
            jax.ShapeDtypeStruct((N, H), _BF),
        ),
    )(o, x, wo, bo, g, b, pos, posW, posb, wq, bq, wk, bk, wv, bv)


def _post_proj_glb_body(o_ref, x_ref, wo_ref, bo_ref, g_ref, b_ref,
                        wq_ref, bq_ref, wk_ref, bk_ref, wv_ref, bv_ref,
                        xn_ref, q_ref, k_ref, v_ref):
    r = x_ref[...] + _dot(o_ref[...], wo_ref[...]) + bo_ref[...]
    xn = _ln(r, g_ref[...], b_ref[...])
    xn_ref[...] = xn
    _qkv(xn, wq_ref, bq_ref, wk_ref, bk_ref, wv_ref, bv_ref,
         q_ref, k_ref, v_ref)


def _post_proj_glb(o, x, wo, bo, g, b, wq, bq, wk, bk, wv, bv):
    return pl.pallas_call(
        _post_proj_glb_body,
        out_shape=(
            jax.ShapeDtypeStruct((N, H), _F32),
            jax.ShapeDtypeStruct((N, H), _BF),
            jax.ShapeDtypeStruct((N, H), _BF),
            jax.ShapeDtypeStruct((N, H), _BF),
        ),
    )(o, x, wo, bo, g, b, wq, bq, wk, bk, wv, bv)


def _post_final_body(o_ref, x_ref, wo_ref, bo_ref, g_ref, b_ref,
                     ow_ref, ob_ref, out_ref):
    r = x_ref[...] + _dot(o_ref[...], wo_ref[...]) + bo_ref[...]
    xn = _ln(r, g_ref[...], b_ref[...])
    out_ref[...] = _dot(xn, ow_ref[...]) + ob_ref[...]


def _post_final(o, x, wo, bo, g, b, ow, ob):
    return pl.pallas_call(
        _post_final_body,
        out_shape=jax.ShapeDtypeStruct((N, OUT), _F32),
    )(o, x, wo, bo, g, b, ow, ob)


_E = 32768
_NW = 32
_EPW = _E // _NW
_CH = 512
_ROWS = N * N // 128
_BAND = 4096
_BSL = _BAND // 16
_DUMP = 16


def _sc_cnt_body(ei0_hbm, ei1_hbm, et_hbm, z_hbm, out_hbm,
                 ei0b, ei1b, etb, rowb, laneb, valb, vals, idx4, acc):
    c = lax.axis_index("c")
    sub = lax.axis_index("s")
    wid = c * 16 + sub
    base = wid * _EPW
    pltpu.sync_copy(ei0_hbm.at[pl.ds(base, _EPW)], ei0b)
    pltpu.sync_copy(ei1_hbm.at[pl.ds(base, _EPW)], ei1b)
    pltpu.sync_copy(et_hbm.at[pl.ds(base, _EPW)], etb)
    pltpu.sync_copy(z_hbm.at[pl.ds(0, _CH)], vals)
    lanes16 = lax.iota(jnp.int32, 16)
    for j in range(_EPW // 16):
        sl = pl.ds(j * 16, 16)
        e0 = ei0b[sl]
        e1 = ei1b[sl]
        et = etb[sl]
        rowb[sl] = e0 * 16 + lax.shift_right_logical(e1, 7)
        laneb[sl] = lax.bitwise_and(e1, 127)
        valb[sl] = lax.shift_left(jnp.int32(1), et * 4).astype(_F32)
    for band_i in range(4):
        b = band_i * 2 + c
        lo = b * _BAND
        pltpu.sync_copy(z_hbm.at[pl.ds(sub * _BSL, _BSL)],
                        acc.at[pl.ds(sub * _BSL, _BSL)])
        plsc.subcore_barrier()
        for ch in range(_EPW // _CH):
            eb = ch * _CH
            for j in range(_CH // 16):
                sl = pl.ds(eb + j * 16, 16)
                eidx = j * 16 + lanes16
                plsc.store_scatter(vals, [eidx, laneb[sl]], valb[sl])
                local = rowb[sl] - lo
                ok = (local >= 0) & (local < _BAND)
                sel = jnp.where(ok, local, _BAND + sub)
                idx4[j // 8, pl.ds((j % 8) * 16, 16)] = sel
            for jj in range(_CH // 128):
                pltpu.sync_copy(vals.at[pl.ds(jj * 128, 128)],
                                acc.at[idx4.at[jj]], add=True)
            zv = jnp.zeros((16,), _F32)
            for j in range(_CH // 16):
                sl = pl.ds(eb + j * 16, 16)
                eidx = j * 16 + lanes16
                plsc.store_scatter(vals, [eidx, laneb[sl]], zv)
        plsc.subcore_barrier()
        pltpu.sync_copy(acc.at[pl.ds(sub * _BSL, _BSL)],
                        out_hbm.at[pl.ds(lo + sub * _BSL, _BSL)])
        plsc.subcore_barrier()


def _build_cnt(edge_index, edge_types):
    cp = pltpu.CompilerParams()
    if "needs_layout_passes" in pltpu.CompilerParams.__dataclass_fields__:
        cp = dataclasses.replace(cp, needs_layout_passes=False)
    mesh = plsc.VectorSubcoreMesh(core_axis_name="c", subcore_axis_name="s")
    zeros2 = jnp.zeros((_BAND, 128), _F32)
    sc = pl.kernel(
        _sc_cnt_body,
        mesh=mesh,
        out_type=jax.ShapeDtypeStruct((_ROWS, 128), _F32),
        scratch_types=[
            pltpu.VMEM((_EPW,), jnp.int32),
            pltpu.VMEM((_EPW,), jnp.int32),
            pltpu.VMEM((_EPW,), jnp.int32),
            pltpu.VMEM((_EPW,), jnp.int32),
            pltpu.VMEM((_EPW,), jnp.int32),
            pltpu.VMEM((_EPW,), _F32),
            pltpu.VMEM((_CH, 128), _F32),
            pltpu.VMEM((4, 128), jnp.int32),
            pltpu.VMEM_SHARED((_BAND + _DUMP, 128), _F32),
        ],
        compiler_params=cp,
    )
    flat = sc(edge_index[0], edge_index[1], edge_types, zeros2)
    return flat.reshape(N, N)


def _r2(v):
    return v.reshape(1, -1)


def kernel(x, edge_index, edge_types, pos_encoding, community_ids, adj_matrix,
           emb_W, emb_b,
           loc0_posW, loc0_posb, loc0_Wq, loc0_bq, loc0_Wk, loc0_bk,
           loc0_Wv, loc0_bv, loc0_Wo, loc0_bo, loc0_ebias, loc0_lng, loc0_lnb,
           loc1_posW, loc1_posb, loc1_Wq, loc1_bq, loc1_Wk, loc1_bk,
           loc1_Wv, loc1_bv, loc1_Wo, loc1_bo, loc1_ebias, loc1_lng, loc1_lnb,
           glb0_Wq, glb0_bq, glb0_Wk, glb0_bk, glb0_Wv, glb0_bv,
           glb0_Wo, glb0_bo, glb0_lng, glb0_lnb,
           glb1_Wq, glb1_bq, glb1_Wk, glb1_bk, glb1_Wv, glb1_bv,
           glb1_Wo, glb1_bo, glb1_lng, glb1_lnb,
           out_W, out_b):
    cnt = _build_cnt(edge_index, edge_types)
    cid_r3 = community_ids.reshape(NI, TI, 1)
    cid_c = community_ids.reshape(1, N)

    x0, q, k, v = _embed_proj(x, emb_W, _r2(emb_b), pos_encoding, loc0_posW,
                              _r2(loc0_posb), loc0_Wq, _r2(loc0_bq),
                              loc0_Wk, _r2(loc0_bk), loc0_Wv, _r2(loc0_bv))
    o = _loc_attn(q, k, v, cnt, loc0_ebias.T, cid_r3, cid_c)
    x1, q, k, v = _post_proj_loc(o, x0, loc0_Wo, _r2(loc0_bo), _r2(loc0_lng),
                                 _r2(loc0_lnb), pos_encoding, loc1_posW,
                                 _r2(loc1_posb), loc1_Wq, _r2(loc1_bq),
                                 loc1_Wk, _r2(loc1_bk), loc1_Wv, _r2(loc1_bv))
    o = _loc_attn(q, k, v, cnt, loc1_ebias.T, cid_r3, cid_c)
    x2, q, k, v = _post_proj_glb(o, x1, loc1_Wo, _r2(loc1_bo), _r2(loc1_lng),
                                 _r2(loc1_lnb), glb0_Wq, _r2(glb0_bq),
                                 glb0_Wk, _r2(glb0_bk), glb0_Wv, _r2(glb0_bv))
    o = _glb_attn(q, k, v, cid_r3, cid_c)
    x3, q, k, v = _post_proj_glb(o, x2, glb0_Wo, _r2(glb0_bo), _r2(glb0_lng),
                                 _r2(glb0_lnb), glb1_Wq, _r2(glb1_bq),
                                 glb1_Wk, _r2(glb1_bk), glb1_Wv, _r2(glb1_bv))
    o = _glb_attn(q, k, v, cid_r3, cid_c)
    return _post_final(o, x3, glb1_Wo, _r2(glb1_bo), _r2(glb1_lng),
                       _r2(glb1_lnb), out_W, _r2(out_b))
